# baseline (speedup 1.0000x reference)
import functools
import math

import jax
import jax.numpy as jnp
from jax import lax
from jax.experimental import pallas as pl
from jax.experimental.pallas import tpu as pltpu
from jax.experimental.pallas import tpu_sc as plsc

_BB = 128
_NBUF = 4


@functools.lru_cache(maxsize=None)
def _build(B, L, V, D):
    info = plsc.get_sparse_core_info()
    NC, NS = info.num_cores, info.num_subcores
    NW = NC * NS
    scale = math.sqrt(D)
    mesh = plsc.VectorSubcoreMesh(core_axis_name="c", subcore_axis_name="s")

    @functools.partial(
        pl.kernel,
        mesh=mesh,
        out_type=jax.ShapeDtypeStruct((L, D, B), jnp.float32),
        compiler_params=pltpu.CompilerParams(
            use_tc_tiling_on_sc=True, needs_layout_passes=False
        ),
        scratch_types=(
            [pltpu.VMEM((L, _BB), jnp.int32)]
            + [pltpu.VMEM((_BB,), jnp.int32) for _ in range(_NBUF)]
            + [pltpu.VMEM((_BB, 128), jnp.float32) for _ in range(_NBUF)]
            + [pltpu.VMEM((D, _BB), jnp.float32)]
            + [pltpu.SemaphoreType.DMA for _ in range(_NBUF)]
        ),
    )
    def emb(idx_hbm, w_hbm, out_hbm, idx_v, *rest):
        rbufs = rest[:_NBUF]
        gbufs = rest[_NBUF:2 * _NBUF]
        tbuf = rest[2 * _NBUF]
        sems = rest[2 * _NBUF + 1:]
        wid = lax.axis_index("s") * NC + lax.axis_index("c")
        b0 = wid * _BB

        pltpu.sync_copy(idx_hbm.at[wid], idx_v)

        def issue(l, slot):
            for t in range(_BB // 16):
                sl = pl.ds(t * 16, 16)
                rbufs[slot][sl] = lax.shift_right_logical(idx_v[l, sl], 1)
            pltpu.async_copy(w_hbm.at[rbufs[slot]], gbufs[slot], sems[slot])

        for b in range(_NBUF):
            issue(b, b)

        iota = lax.broadcasted_iota(jnp.int32, (16,), 0)

        def outer(o, carry):
            for b in range(_NBUF):
                l = o * _NBUF + b
                pltpu.make_async_copy(
                    w_hbm.at[rbufs[b]], gbufs[b], sems[b]
                ).wait()

                rows = []
                cbase = []
                for j in range(_BB // 16):
                    rows.append(iota + (16 * j))
                    pj = lax.bitwise_and(idx_v[l, pl.ds(16 * j, 16)], 1)
                    cbase.append(lax.shift_left(pj, 6))

                def col(c, carry2, _gb=gbufs[b]):
                    for j in range(_BB // 16):
                        v = plsc.load_gather(_gb, [rows[j], cbase[j] + c])
                        tbuf[c, pl.ds(16 * j, 16)] = v * scale
                    return carry2

                lax.fori_loop(0, D, col, 0)

                pltpu.sync_copy(tbuf, out_hbm.at[l, :, pl.ds(b0, _BB)])

                nxt = l + _NBUF

                @pl.when(nxt < L)
                def _():
                    issue(nxt, b)

            return carry

        lax.fori_loop(0, L // _NBUF, outer, 0)

    return emb


def kernel(x, W):
    B, L = x.shape
    V, D = W.shape
    NWB = B // _BB
    xt = jnp.transpose(x.reshape(NWB, _BB, L), (0, 2, 1)).astype(jnp.int32)
    wr = W.reshape(V // 2, 2 * D)
    out = _build(B, L, V, D)(xt, wr)
    return jnp.transpose(out, (2, 0, 1))


# speedup vs baseline: 1.1950x; 1.1950x over previous
import functools
import math

import jax
import jax.numpy as jnp
from jax import lax
from jax.experimental import pallas as pl
from jax.experimental.pallas import tpu as pltpu
from jax.experimental.pallas import tpu_sc as plsc

_CHUNK = 128
_NBUF = 4
_TBLK = 512


@functools.lru_cache(maxsize=None)
def _build_prep(V, D):
    scale = math.sqrt(D)
    grid = (V + _TBLK - 1) // _TBLK

    def prep(wt_ref, o_ref):
        o_ref[:, :D] = jnp.transpose(wt_ref[...], (1, 0)) * scale
        o_ref[:, D:] = jnp.zeros((_TBLK, D), jnp.float32)

    return pl.pallas_call(
        prep,
        grid=(grid,),
        in_specs=[pl.BlockSpec((D, _TBLK), lambda i: (0, i))],
        out_specs=pl.BlockSpec((_TBLK, 2 * D), lambda i: (i, 0)),
        out_shape=jax.ShapeDtypeStruct((V, 2 * D), jnp.float32),
    )


@functools.lru_cache(maxsize=None)
def _build_gather(total, V, D, n_chunks):
    info = plsc.get_sparse_core_info()
    NC, NS = info.num_cores, info.num_subcores
    NW = NC * NS
    per_w = total // NW
    mesh = plsc.VectorSubcoreMesh(core_axis_name="c", subcore_axis_name="s")

    @functools.partial(
        pl.kernel,
        mesh=mesh,
        out_type=jax.ShapeDtypeStruct((total, 2 * D), jnp.float32),
        compiler_params=pltpu.CompilerParams(
            use_tc_tiling_on_sc=True, needs_layout_passes=False
        ),
        scratch_types=(
            [pltpu.VMEM((n_chunks, _CHUNK), jnp.int32)]
            + [pltpu.VMEM((_CHUNK,), jnp.int32) for _ in range(_NBUF)]
            + [pltpu.VMEM((_CHUNK, 2 * D), jnp.float32) for _ in range(_NBUF)]
            + [pltpu.SemaphoreType.DMA for _ in range(_NBUF)]
        ),
    )
    def emb(idx_hbm, w_hbm, out_hbm, idx_v, *rest):
        rbufs = rest[:_NBUF]
        gbufs = rest[_NBUF:2 * _NBUF]
        sems = rest[2 * _NBUF:]
        wid = lax.axis_index("s") * NC + lax.axis_index("c")
        row0 = wid * per_w

        pltpu.sync_copy(idx_hbm.at[wid], idx_v)

        def issue(chunk, slot):
            for t in range(_CHUNK // 16):
                sl = pl.ds(t * 16, 16)
                rbufs[slot][sl] = idx_v[chunk, sl]
            pltpu.async_copy(w_hbm.at[rbufs[slot]], gbufs[slot], sems[slot])

        for b in range(_NBUF):
            issue(b, b)

        def outer(o, carry):
            for b in range(_NBUF):
                chunk = o * _NBUF + b
                pltpu.make_async_copy(
                    w_hbm.at[rbufs[b]], gbufs[b], sems[b]
                ).wait()

                pltpu.sync_copy(
                    gbufs[b],
                    out_hbm.at[pl.ds(row0 + chunk * _CHUNK, _CHUNK)],
                )

                nxt = chunk + _NBUF

                @pl.when(nxt < n_chunks)
                def _():
                    issue(nxt, b)

            return carry

        lax.fori_loop(0, n_chunks // _NBUF, outer, 0)

    return emb


def kernel(x, W):
    B, L = x.shape
    V, D = W.shape
    total = B * L
    info = plsc.get_sparse_core_info()
    NW = info.num_cores * info.num_subcores
    per_w = total // NW
    n_chunks = per_w // _CHUNK
    idx = x.reshape(NW, n_chunks, _CHUNK).astype(jnp.int32)
    wp = _build_prep(V, D)(W.T)
    out = _build_gather(total, V, D, n_chunks)(idx, wp)
    return out.reshape(B, L, 2 * D)[:, :, :D]


# speedup vs baseline: 2.6568x; 2.2232x over previous
import functools
import math

import jax
import jax.numpy as jnp
from jax import lax
from jax.experimental import pallas as pl
from jax.experimental.pallas import tpu as pltpu
from jax.experimental.pallas import tpu_sc as plsc

_CHUNK = 128
_NBUF = 4
_TBLK = 8192


@functools.lru_cache(maxsize=None)
def _build_prep(V, D):
    scale = math.sqrt(D)
    grid = (V + _TBLK - 1) // _TBLK

    def prep(wt_ref, o_ref):
        eye = jnp.eye(D, 2 * D, dtype=jnp.float32) * scale
        o_ref[...] = lax.dot_general(
            wt_ref[...], eye, (((0,), (0,)), ((), ())),
            precision=lax.Precision.DEFAULT,
            preferred_element_type=jnp.float32,
        )

    return pl.pallas_call(
        prep,
        grid=(grid,),
        in_specs=[pl.BlockSpec((D, _TBLK), lambda i: (0, i))],
        out_specs=pl.BlockSpec((_TBLK, 2 * D), lambda i: (i, 0)),
        out_shape=jax.ShapeDtypeStruct((V, 2 * D), jnp.float32),
    )


@functools.lru_cache(maxsize=None)
def _build_gather(total, V, D, n_chunks):
    info = plsc.get_sparse_core_info()
    NC, NS = info.num_cores, info.num_subcores
    NW = NC * NS
    per_w = total // NW
    mesh = plsc.VectorSubcoreMesh(core_axis_name="c", subcore_axis_name="s")

    @functools.partial(
        pl.kernel,
        mesh=mesh,
        out_type=jax.ShapeDtypeStruct((total, 2 * D), jnp.float32),
        compiler_params=pltpu.CompilerParams(
            use_tc_tiling_on_sc=True, needs_layout_passes=False
        ),
        scratch_types=(
            [pltpu.VMEM((n_chunks, _CHUNK), jnp.int32)]
            + [pltpu.VMEM((_CHUNK,), jnp.int32) for _ in range(_NBUF)]
            + [pltpu.VMEM((_CHUNK, 2 * D), jnp.float32) for _ in range(_NBUF)]
            + [pltpu.SemaphoreType.DMA for _ in range(_NBUF)]
        ),
    )
    def emb(idx_hbm, w_hbm, out_hbm, idx_v, *rest):
        rbufs = rest[:_NBUF]
        gbufs = rest[_NBUF:2 * _NBUF]
        sems = rest[2 * _NBUF:]
        wid = lax.axis_index("s") * NC + lax.axis_index("c")
        row0 = wid * per_w

        pltpu.sync_copy(idx_hbm.at[wid], idx_v)

        def issue(chunk, slot):
            for t in range(_CHUNK // 16):
                sl = pl.ds(t * 16, 16)
                rbufs[slot][sl] = idx_v[chunk, sl]
            pltpu.async_copy(w_hbm.at[rbufs[slot]], gbufs[slot], sems[slot])

        for b in range(_NBUF):
            issue(b, b)

        def outer(o, carry):
            for b in range(_NBUF):
                chunk = o * _NBUF + b
                pltpu.make_async_copy(
                    w_hbm.at[rbufs[b]], gbufs[b], sems[b]
                ).wait()

                pltpu.sync_copy(
                    gbufs[b],
                    out_hbm.at[pl.ds(row0 + chunk * _CHUNK, _CHUNK)],
                )

                nxt = chunk + _NBUF

                @pl.when(nxt < n_chunks)
                def _():
                    issue(nxt, b)

            return carry

        lax.fori_loop(0, n_chunks // _NBUF, outer, 0)

    return emb


def kernel(x, W):
    B, L = x.shape
    V, D = W.shape
    total = B * L
    info = plsc.get_sparse_core_info()
    NW = info.num_cores * info.num_subcores
    per_w = total // NW
    n_chunks = per_w // _CHUNK
    idx = x.reshape(NW, n_chunks, _CHUNK).astype(jnp.int32)
    wp = _build_prep(V, D)(W.T)
    out = _build_gather(total, V, D, n_chunks)(idx, wp)
    return out.reshape(B, L, 2 * D)[:, :, :D]
